# baseline (speedup 1.0000x reference)
import dataclasses

import jax
import jax.numpy as jnp
from jax import lax
from jax.experimental import pallas as pl
from jax.experimental.pallas import tpu as pltpu
from jax.experimental.pallas import tpu_sc as plsc

_VOCAB = 1000
_VPAD = 1024
_DIM = 128
_HIST = 200
_NC = 2
_NS = 16
_L = 16
_NW = _NC * _NS
_G = 16
_TB = 1024
_MB = 1024


def _hist_body(idx_hbm, counts_hbm, idx_bufs, bins_bufs, sems_in, sems_out):
    n_groups = idx_hbm.shape[0] // _NW
    wid = lax.axis_index("s") * _NC + lax.axis_index("c")
    gbase = wid * n_groups
    ones = jnp.ones((_L,), jnp.float32)
    zeros = jnp.zeros((_L,), jnp.float32)

    def row0(g):
        return (gbase + g) * _G

    def out_dma(g, bins_v, sem_out, wait):
        for r in range(_G):
            cp = pltpu.make_async_copy(
                bins_v.at[pl.ds(r * _VPAD, _VPAD)],
                counts_hbm.at[row0(g) + r],
                sem_out,
            )
            cp.wait() if wait else cp.start()

    for bb in range(2):
        @pl.loop(0, _G * _VPAD, step=_L, unroll=8)
        def _zero(i, _bins=bins_bufs[bb]):
            _bins[pl.ds(i, _L)] = zeros

    pltpu.async_copy(idx_hbm.at[gbase], idx_bufs[0], sems_in[0])
    pltpu.async_copy(idx_hbm.at[gbase + 1], idx_bufs[1], sems_in[1])

    def body(g, ib, bb):
        idx_v, sem_in = idx_bufs[ib], sems_in[ib]
        idx_prev = idx_bufs[(ib + 2) % 4]
        bins_v, sem_out = bins_bufs[bb], sems_out[bb]
        ib_next = (ib + 2) % 4

        @pl.when(g >= 2)
        def _retire():
            out_dma(g - 2, bins_v, sem_out, wait=True)

            @pl.loop(0, _HIST, unroll=8)
            def _scatter_zero(t):
                idxs = idx_prev[pl.ds(t * _G, _G)]
                plsc.store_scatter(bins_v, [idxs], zeros)

        @pl.when(g + 2 < n_groups)
        def _prefetch():
            pltpu.async_copy(
                idx_hbm.at[gbase + g + 2],
                idx_bufs[ib_next],
                sems_in[ib_next],
            )

        pltpu.make_async_copy(idx_hbm.at[gbase + g], idx_v, sem_in).wait()

        @pl.loop(0, _HIST, unroll=8)
        def _scatter(t):
            idxs = idx_v[pl.ds(t * _G, _G)]
            plsc.addupdate_scatter(bins_v, [idxs], ones)

        out_dma(g, bins_v, sem_out, wait=False)

    @pl.loop(0, n_groups, step=4)
    def _group(g):
        body(g, 0, 0)
        body(g + 1, 1, 1)
        body(g + 2, 2, 0)
        body(g + 3, 3, 1)

    out_dma(n_groups - 2, bins_bufs[0], sems_out[0], wait=True)
    out_dma(n_groups - 1, bins_bufs[1], sems_out[1], wait=True)


def _tr_body(idx_ref, out_ref):
    bias = lax.broadcasted_iota(jnp.int32, (_TB // _G, _G, _HIST), 1) * _VPAD
    x = idx_ref[...].reshape(_TB // _G, _G, _HIST) + bias
    out_ref[...] = x.transpose(0, 2, 1).reshape(_TB // _G, _HIST * _G)


def _mm_body(cnt_ref, ehi_ref, elo_ref, out_ref):
    cb = cnt_ref[...].astype(jnp.bfloat16)
    r = jnp.dot(cb, ehi_ref[...], preferred_element_type=jnp.float32)
    r = r + jnp.dot(cb, elo_ref[...], preferred_element_type=jnp.float32)
    out_ref[...] = jnp.maximum(r * (1.0 / _HIST), 0.0)


def kernel(artists_batch, embedding_weight):
    batch = artists_batch.shape[0]

    idx_t = pl.pallas_call(
        _tr_body,
        grid=(batch // _TB,),
        in_specs=[pl.BlockSpec((_TB, _HIST), lambda i: (i, 0))],
        out_specs=pl.BlockSpec((_TB // _G, _HIST * _G), lambda i: (i, 0)),
        out_shape=jax.ShapeDtypeStruct((batch // _G, _HIST * _G), jnp.int32),
    )(artists_batch)

    sc_params = pltpu.CompilerParams()
    if "needs_layout_passes" in pltpu.CompilerParams.__dataclass_fields__:
        sc_params = dataclasses.replace(sc_params, needs_layout_passes=False)
    mesh = plsc.VectorSubcoreMesh(core_axis_name="c", subcore_axis_name="s")
    counts = pl.kernel(
        _hist_body,
        out_type=jax.ShapeDtypeStruct((batch, _VPAD), jnp.float32),
        mesh=mesh,
        scratch_types=[
            [pltpu.VMEM((_HIST * _G,), jnp.int32) for _ in range(4)],
            [pltpu.VMEM((_G * _VPAD,), jnp.float32) for _ in range(2)],
            [pltpu.SemaphoreType.DMA for _ in range(4)],
            [pltpu.SemaphoreType.DMA for _ in range(2)],
        ],
        compiler_params=sc_params,
    )(idx_t)

    ew = jnp.pad(embedding_weight, ((0, _VPAD - _VOCAB), (0, 0)))
    ehi = ew.astype(jnp.bfloat16)
    elo = (ew - ehi.astype(jnp.float32)).astype(jnp.bfloat16)
    return pl.pallas_call(
        _mm_body,
        grid=(batch // _MB,),
        in_specs=[
            pl.BlockSpec((_MB, _VPAD), lambda i: (i, 0)),
            pl.BlockSpec((_VPAD, _DIM), lambda i: (0, 0)),
            pl.BlockSpec((_VPAD, _DIM), lambda i: (0, 0)),
        ],
        out_specs=pl.BlockSpec((_MB, _DIM), lambda i: (i, 0)),
        out_shape=jax.ShapeDtypeStruct((batch, _DIM), jnp.float32),
    )(counts, ehi, elo)


# speedup vs baseline: 1.0990x; 1.0990x over previous
import dataclasses

import jax
import jax.numpy as jnp
from jax import lax
from jax.experimental import pallas as pl
from jax.experimental.pallas import tpu as pltpu
from jax.experimental.pallas import tpu_sc as plsc

_VOCAB = 1000
_VPAD = 1024
_DIM = 128
_HIST = 200
_NC = 2
_NS = 16
_L = 16
_NW = _NC * _NS
_G = 16
_MB = 1024


def _hist_body(idx_hbm, counts_hbm, idx_bufs, bins_bufs, sems_in, sems_out):
    n_groups = idx_hbm.shape[0] // _NW
    wid = lax.axis_index("s") * _NC + lax.axis_index("c")
    gbase = wid * n_groups
    ones = jnp.ones((_L,), jnp.float32)
    zeros = jnp.zeros((_L,), jnp.float32)

    def row0(g):
        return (gbase + g) * _G

    def out_dma(g, bins_v, sem_out, wait):
        for r in range(_G):
            cp = pltpu.make_async_copy(
                bins_v.at[pl.ds(r * _VPAD, _VPAD)],
                counts_hbm.at[row0(g) + r],
                sem_out,
            )
            cp.wait() if wait else cp.start()

    for bb in range(2):
        @pl.loop(0, _G * _VPAD, step=_L, unroll=8)
        def _zero(i, _bins=bins_bufs[bb]):
            _bins[pl.ds(i, _L)] = zeros

    pltpu.async_copy(idx_hbm.at[gbase], idx_bufs[0], sems_in[0])
    pltpu.async_copy(idx_hbm.at[gbase + 1], idx_bufs[1], sems_in[1])

    def body(g, ib, bb):
        idx_v, sem_in = idx_bufs[ib], sems_in[ib]
        idx_prev = idx_bufs[(ib + 2) % 4]
        bins_v, sem_out = bins_bufs[bb], sems_out[bb]
        ib_next = (ib + 2) % 4

        @pl.when(g >= 2)
        def _retire():
            out_dma(g - 2, bins_v, sem_out, wait=True)

            @pl.loop(0, _HIST, unroll=8)
            def _scatter_zero(t):
                idxs = idx_prev[pl.ds(t * _G, _G)]
                plsc.store_scatter(bins_v, [idxs], zeros)

        @pl.when(g + 2 < n_groups)
        def _prefetch():
            pltpu.async_copy(
                idx_hbm.at[gbase + g + 2],
                idx_bufs[ib_next],
                sems_in[ib_next],
            )

        pltpu.make_async_copy(idx_hbm.at[gbase + g], idx_v, sem_in).wait()

        @pl.loop(0, _HIST, unroll=8)
        def _scatter(t):
            idxs = idx_v[pl.ds(t * _G, _G)]
            plsc.addupdate_scatter(bins_v, [idxs], ones)

        out_dma(g, bins_v, sem_out, wait=False)

    @pl.loop(0, n_groups, step=4)
    def _group(g):
        body(g, 0, 0)
        body(g + 1, 1, 1)
        body(g + 2, 2, 0)
        body(g + 3, 3, 1)

    out_dma(n_groups - 2, bins_bufs[0], sems_out[0], wait=True)
    out_dma(n_groups - 1, bins_bufs[1], sems_out[1], wait=True)


def _mm_body(cnt_ref, ehi_ref, elo_ref, out_ref):
    cb = cnt_ref[...].astype(jnp.bfloat16)
    r = jnp.dot(cb, ehi_ref[...], preferred_element_type=jnp.float32)
    r = r + jnp.dot(cb, elo_ref[...], preferred_element_type=jnp.float32)
    out_ref[...] = jnp.maximum(r * (1.0 / _HIST), 0.0)


def kernel(artists_batch, embedding_weight):
    batch = artists_batch.shape[0]

    lane_bias = (jnp.arange(_G, dtype=jnp.int32) * _VPAD)[None, :, None]
    idx_t = (
        (artists_batch.reshape(batch // _G, _G, _HIST) + lane_bias)
        .transpose(0, 2, 1)
        .reshape(batch // _G, _HIST * _G)
    )

    sc_params = pltpu.CompilerParams()
    if "needs_layout_passes" in pltpu.CompilerParams.__dataclass_fields__:
        sc_params = dataclasses.replace(sc_params, needs_layout_passes=False)
    mesh = plsc.VectorSubcoreMesh(core_axis_name="c", subcore_axis_name="s")
    counts = pl.kernel(
        _hist_body,
        out_type=jax.ShapeDtypeStruct((batch, _VPAD), jnp.float32),
        mesh=mesh,
        scratch_types=[
            [pltpu.VMEM((_HIST * _G,), jnp.int32) for _ in range(4)],
            [pltpu.VMEM((_G * _VPAD,), jnp.float32) for _ in range(2)],
            [pltpu.SemaphoreType.DMA for _ in range(4)],
            [pltpu.SemaphoreType.DMA for _ in range(2)],
        ],
        compiler_params=sc_params,
    )(idx_t)

    ew = jnp.pad(embedding_weight, ((0, _VPAD - _VOCAB), (0, 0)))
    ehi = ew.astype(jnp.bfloat16)
    elo = (ew - ehi.astype(jnp.float32)).astype(jnp.bfloat16)
    return pl.pallas_call(
        _mm_body,
        grid=(batch // _MB,),
        in_specs=[
            pl.BlockSpec((_MB, _VPAD), lambda i: (i, 0)),
            pl.BlockSpec((_VPAD, _DIM), lambda i: (0, 0)),
            pl.BlockSpec((_VPAD, _DIM), lambda i: (0, 0)),
        ],
        out_specs=pl.BlockSpec((_MB, _DIM), lambda i: (i, 0)),
        out_shape=jax.ShapeDtypeStruct((batch, _DIM), jnp.float32),
    )(counts, ehi, elo)
